# Initial kernel scaffold; baseline (speedup 1.0000x reference)
import jax
import jax.numpy as jnp
from jax.experimental import pallas as pl


def kernel(view, command, args, view_embed_w, command_embed_w, args_embed_w, args_mlp_w, args_mlp_b, mlp_w, mlp_b, pos_embed_w):
    raise NotImplementedError("write your pallas kernel here")


# speedup vs baseline: 4.0571x; 4.0571x over previous
import functools

import jax
import jax.numpy as jnp
from jax import lax
from jax.experimental import pallas as pl
from jax.experimental.pallas import tpu as pltpu
from jax.experimental.pallas import tpu_sc as plsc

D_MODEL = 256
N_ARGS = 11
S, N = 512, 512
VC_ROWS = 32
TBL_STRIDE = 264
TBL_ROWS = VC_ROWS + N_ARGS * TBL_STRIDE
NC, NS = 2, 16
NW = NC * NS
ROWS_PER_W = S // NW
CHUNK = 32
N_GATHER = 12


def _build_tables_body(view_w_ref, cmd_w_ref, args_w_ref, args_mlp_w_ref,
                       args_mlp_b_ref, mlp_w_ref, mlp_b_ref, pos_w_ref,
                       tbl_ref, pos_ref):
    f32 = jnp.float32
    hi = jax.lax.Precision.HIGHEST
    mw_v = mlp_w_ref[0:4, :]
    mw_c = mlp_w_ref[4:12, :]
    mw_a = mlp_w_ref[12:140, :]
    vt = jnp.dot(view_w_ref[...], mw_v, precision=hi)
    ct = jnp.dot(cmd_w_ref[...], mw_c, precision=hi)
    r = lax.broadcasted_iota(jnp.int32, (VC_ROWS, 8), 0)
    ohv = (r // 7 == lax.broadcasted_iota(jnp.int32, (VC_ROWS, 8), 1)).astype(f32)
    ohc = (r % 7 == lax.broadcasted_iota(jnp.int32, (VC_ROWS, 8), 1)).astype(f32)
    vt8 = jnp.concatenate([vt, jnp.zeros((4, D_MODEL), f32)], axis=0)
    ct8 = jnp.concatenate([ct, jnp.zeros((1, D_MODEL), f32)], axis=0)
    vc = (jnp.dot(ohv, vt8, precision=hi)
          + jnp.dot(ohc, ct8, precision=hi))
    tbl_ref[0:VC_ROWS, :] = vc
    for j in range(N_ARGS):
        m_j = jnp.dot(args_mlp_w_ref[64 * j:64 * (j + 1), :], mw_a, precision=hi)
        at_j = jnp.dot(args_w_ref[...], m_j, precision=hi)
        base = VC_ROWS + j * TBL_STRIDE
        tbl_ref[base:base + 257, :] = at_j
        tbl_ref[base + 257:base + TBL_STRIDE, :] = jnp.zeros((TBL_STRIDE - 257, D_MODEL), f32)
    bias = mlp_b_ref[...] + jnp.dot(args_mlp_b_ref[...], mw_a, precision=hi)
    pos_ref[...] = pos_w_ref[0:S, :] + bias


def _build_tables(view_w, cmd_w, args_w, args_mlp_w, args_mlp_b, mlp_w, mlp_b, pos_w):
    return pl.pallas_call(
        _build_tables_body,
        out_shape=(
            jax.ShapeDtypeStruct((TBL_ROWS, D_MODEL), jnp.float32),
            jax.ShapeDtypeStruct((S, D_MODEL), jnp.float32),
        ),
    )(view_w, cmd_w, args_w, args_mlp_w,
      args_mlp_b.reshape(1, -1), mlp_w, mlp_b.reshape(1, -1), pos_w)


def _sc_body(view_hbm, cmd_hbm, argst_hbm, tbl_hbm, pos_hbm, out_hbm,
             vrow, crow, arow, idx, posv, gbuf, outv, sem):
    wid = lax.axis_index("s") * NC + lax.axis_index("c")

    def do_row(r, _):
        s = wid * ROWS_PER_W + r
        pltpu.sync_copy(view_hbm.at[pl.ds(s * N, N)], vrow)
        pltpu.sync_copy(cmd_hbm.at[pl.ds(s * N, N)], crow)
        for j in range(N_ARGS):
            pltpu.sync_copy(argst_hbm.at[pl.ds((j * S + s) * N, N)],
                            arow.at[pl.ds(j * N, N)])
        pltpu.sync_copy(pos_hbm.at[pl.ds(s * D_MODEL, D_MODEL)], posv)
        for g in range(N // 16):
            sl = pl.ds(g * 16, 16)
            idx[sl] = vrow[sl] * 7 + crow[sl]
            for j in range(N_ARGS):
                idx[pl.ds((1 + j) * N + g * 16, 16)] = (
                    arow[pl.ds(j * N + g * 16, 16)]
                    + (VC_ROWS + TBL_STRIDE * j + 1))

        def do_chunk(ch, _):
            base = ch * CHUNK
            descs = []
            for k in range(N_GATHER):
                descs.append(pltpu.async_copy(
                    tbl_hbm.at[idx.at[pl.ds(k * N + base, CHUNK)]],
                    gbuf.at[pl.ds(k * CHUNK, CHUNK)], sem))
            for d in descs:
                d.wait()

            def do_tok(t, _):
                for fg in range(D_MODEL // 16):
                    fsl = pl.ds(fg * 16, 16)
                    acc = posv[fsl]
                    for k in range(N_GATHER):
                        acc = acc + gbuf[k * CHUNK + t, fsl]
                    outv[t, fsl] = acc
                return 0

            lax.fori_loop(0, CHUNK, do_tok, 0)
            pltpu.sync_copy(outv, out_hbm.at[pl.ds(s * N + base, CHUNK)])
            return 0

        lax.fori_loop(0, N // CHUNK, do_chunk, 0)
        return 0

    lax.fori_loop(0, ROWS_PER_W, do_row, 0)


def _sc_bag(view, cmd, argst, tbl, pos):
    mesh = plsc.VectorSubcoreMesh(core_axis_name="c", subcore_axis_name="s")
    return pl.kernel(
        _sc_body,
        out_type=jax.ShapeDtypeStruct((S * N, D_MODEL), jnp.float32),
        mesh=mesh,
        scratch_types=[
            pltpu.VMEM((N,), jnp.int32),
            pltpu.VMEM((N,), jnp.int32),
            pltpu.VMEM((N_ARGS * N,), jnp.int32),
            pltpu.VMEM((N_GATHER * N,), jnp.int32),
            pltpu.VMEM((D_MODEL,), jnp.float32),
            pltpu.VMEM((N_GATHER * CHUNK, D_MODEL), jnp.float32),
            pltpu.VMEM((CHUNK, D_MODEL), jnp.float32),
            pltpu.SemaphoreType.DMA,
        ],
    )(view, cmd, argst, tbl, pos)


def kernel(view, command, args, view_embed_w, command_embed_w, args_embed_w,
           args_mlp_w, args_mlp_b, mlp_w, mlp_b, pos_embed_w):
    tbl, pos = _build_tables(view_embed_w, command_embed_w, args_embed_w,
                             args_mlp_w, args_mlp_b, mlp_w, mlp_b, pos_embed_w)
    argst = args.transpose(2, 0, 1).reshape(-1)
    out = _sc_bag(view.reshape(-1), command.reshape(-1), argst, tbl,
                  pos.reshape(-1))
    return out.reshape(S, N, D_MODEL)


# speedup vs baseline: 6.7020x; 1.6519x over previous
import functools

import jax
import jax.numpy as jnp
from jax import lax
from jax.experimental import pallas as pl
from jax.experimental.pallas import tpu as pltpu
from jax.experimental.pallas import tpu_sc as plsc

D_MODEL = 256
N_ARGS = 11
S, N = 512, 512
VC_ROWS = 32
TBL_STRIDE = 264
TBL_ROWS = VC_ROWS + N_ARGS * TBL_STRIDE
NC, NS = 2, 16
NW = NC * NS
ROWS_PER_W = S // NW
CHUNK = 32
N_GATHER = 12
D_WORDS = D_MODEL // 2


def _build_tables_body(view_w_ref, cmd_w_ref, args_w_ref, args_mlp_w_ref,
                       args_mlp_b_ref, mlp_w_ref, mlp_b_ref, pos_w_ref,
                       tbl_ref, pos_ref):
    f32 = jnp.float32
    hi = jax.lax.Precision.HIGHEST
    mw_v = mlp_w_ref[0:4, :]
    mw_c = mlp_w_ref[4:12, :]
    mw_a = mlp_w_ref[12:140, :]
    cc = lax.broadcasted_iota(jnp.int32, (D_MODEL, D_MODEL), 1)
    src = ((cc // 32) * 32) + ((cc % 2) * 16) + ((cc % 32) // 2)
    perm = (lax.broadcasted_iota(jnp.int32, (D_MODEL, D_MODEL), 0)
            == src).astype(f32)
    vt = jnp.dot(view_w_ref[...], mw_v, precision=hi)
    ct = jnp.dot(cmd_w_ref[...], mw_c, precision=hi)
    r = lax.broadcasted_iota(jnp.int32, (VC_ROWS, 8), 0)
    ohv = (r // 7 == lax.broadcasted_iota(jnp.int32, (VC_ROWS, 8), 1)).astype(f32)
    ohc = (r % 7 == lax.broadcasted_iota(jnp.int32, (VC_ROWS, 8), 1)).astype(f32)
    vt8 = jnp.concatenate([vt, jnp.zeros((4, D_MODEL), f32)], axis=0)
    ct8 = jnp.concatenate([ct, jnp.zeros((1, D_MODEL), f32)], axis=0)
    vc = (jnp.dot(ohv, vt8, precision=hi)
          + jnp.dot(ohc, ct8, precision=hi))
    tbl_ref[0:VC_ROWS, :] = jnp.dot(vc, perm, precision=hi).astype(jnp.bfloat16)
    for j in range(N_ARGS):
        m_j = jnp.dot(args_mlp_w_ref[64 * j:64 * (j + 1), :], mw_a, precision=hi)
        at_j = jnp.dot(args_w_ref[...], m_j, precision=hi)
        base = VC_ROWS + j * TBL_STRIDE
        tbl_ref[base:base + 257, :] = jnp.dot(at_j, perm, precision=hi).astype(jnp.bfloat16)
        tbl_ref[base + 257:base + TBL_STRIDE, :] = jnp.zeros(
            (TBL_STRIDE - 257, D_MODEL), jnp.bfloat16)
    bias = mlp_b_ref[...] + jnp.dot(args_mlp_b_ref[...], mw_a, precision=hi)
    pos_ref[...] = pos_w_ref[0:S, :] + bias


def _build_tables(view_w, cmd_w, args_w, args_mlp_w, args_mlp_b, mlp_w, mlp_b, pos_w):
    return pl.pallas_call(
        _build_tables_body,
        out_shape=(
            jax.ShapeDtypeStruct((TBL_ROWS, D_MODEL), jnp.bfloat16),
            jax.ShapeDtypeStruct((S, D_MODEL), jnp.float32),
        ),
    )(view_w, cmd_w, args_w, args_mlp_w,
      args_mlp_b.reshape(1, -1), mlp_w, mlp_b.reshape(1, -1), pos_w)


def _sc_body(view_hbm, cmd_hbm, argst_hbm, tbl_hbm, pos_hbm, out_hbm,
             vrow, crow, arow, idx, posv, gbuf, outv, sem):
    wid = lax.axis_index("s") * NC + lax.axis_index("c")

    def do_row(r, _):
        s = wid * ROWS_PER_W + r
        pltpu.sync_copy(view_hbm.at[pl.ds(s * N, N)], vrow)
        pltpu.sync_copy(cmd_hbm.at[pl.ds(s * N, N)], crow)
        for j in range(N_ARGS):
            pltpu.sync_copy(argst_hbm.at[pl.ds((j * S + s) * N, N)],
                            arow.at[pl.ds(j * N, N)])
        pltpu.sync_copy(pos_hbm.at[pl.ds(s * D_MODEL, D_MODEL)], posv)
        for g in range(N // 16):
            sl = pl.ds(g * 16, 16)
            idx[sl] = vrow[sl] * 7 + crow[sl]
            for j in range(N_ARGS):
                idx[pl.ds((1 + j) * N + g * 16, 16)] = (
                    arow[pl.ds(j * N + g * 16, 16)]
                    + (VC_ROWS + TBL_STRIDE * j + 1))

        def do_chunk(ch, _):
            base = ch * CHUNK
            descs = []
            for k in range(N_GATHER):
                descs.append(pltpu.async_copy(
                    tbl_hbm.at[idx.at[pl.ds(k * N + base, CHUNK)]],
                    gbuf.at[pl.ds(k * CHUNK, CHUNK)], sem))
            for d in descs:
                d.wait()

            def do_tok(t, _):
                for wg in range(D_WORDS // 16):
                    wsl = pl.ds(wg * 16, 16)
                    acc_lo = posv[pl.ds(wg * 32, 16)]
                    acc_hi = posv[pl.ds(wg * 32 + 16, 16)]
                    for k in range(N_GATHER):
                        xw = gbuf[k * CHUNK + t, wsl]
                        acc_lo = acc_lo + plsc.bitcast(xw << 16, jnp.float32)
                        acc_hi = acc_hi + plsc.bitcast(
                            xw & jnp.int32(-65536), jnp.float32)
                    outv[t, pl.ds(wg * 32, 16)] = acc_lo
                    outv[t, pl.ds(wg * 32 + 16, 16)] = acc_hi
                return 0

            lax.fori_loop(0, CHUNK, do_tok, 0)
            pltpu.sync_copy(outv, out_hbm.at[pl.ds(s * N + base, CHUNK)])
            return 0

        lax.fori_loop(0, N // CHUNK, do_chunk, 0)
        return 0

    lax.fori_loop(0, ROWS_PER_W, do_row, 0)


def _sc_bag(view, cmd, argst, tbl, pos):
    mesh = plsc.VectorSubcoreMesh(core_axis_name="c", subcore_axis_name="s")
    return pl.kernel(
        _sc_body,
        out_type=jax.ShapeDtypeStruct((S * N, D_MODEL), jnp.float32),
        mesh=mesh,
        compiler_params=pltpu.CompilerParams(needs_layout_passes=False),
        scratch_types=[
            pltpu.VMEM((N,), jnp.int32),
            pltpu.VMEM((N,), jnp.int32),
            pltpu.VMEM((N_ARGS * N,), jnp.int32),
            pltpu.VMEM((N_GATHER * N,), jnp.int32),
            pltpu.VMEM((D_MODEL,), jnp.float32),
            pltpu.VMEM((N_GATHER * CHUNK, D_WORDS), jnp.int32),
            pltpu.VMEM((CHUNK, D_MODEL), jnp.float32),
            pltpu.SemaphoreType.DMA,
        ],
    )(view, cmd, argst, tbl, pos)


def kernel(view, command, args, view_embed_w, command_embed_w, args_embed_w,
           args_mlp_w, args_mlp_b, mlp_w, mlp_b, pos_embed_w):
    tbl, pos = _build_tables(view_embed_w, command_embed_w, args_embed_w,
                             args_mlp_w, args_mlp_b, mlp_w, mlp_b, pos_embed_w)
    argst = args.transpose(2, 0, 1).reshape(-1)
    tbl_i32 = lax.bitcast_convert_type(
        tbl.reshape(TBL_ROWS, D_WORDS, 2), jnp.int32)
    out = _sc_bag(view.reshape(-1), command.reshape(-1), argst, tbl_i32,
                  pos.reshape(-1))
    return out.reshape(S, N, D_MODEL)


# speedup vs baseline: 9.5901x; 1.4309x over previous
import functools

import jax
import jax.numpy as jnp
from jax import lax
from jax.experimental import pallas as pl
from jax.experimental.pallas import tpu as pltpu
from jax.experimental.pallas import tpu_sc as plsc

D_MODEL = 256
N_ARGS = 11
S, N = 512, 512
VC_ROWS = 32
TBL_STRIDE = 264
TBL_ROWS = VC_ROWS + N_ARGS * TBL_STRIDE
NC, NS = 2, 16
NW = NC * NS
ROWS_PER_W = S // NW
CHUNK = 32
N_GATHER = 12
D_WORDS = D_MODEL // 2


def _build_tables_body(view_w_ref, cmd_w_ref, args_w_ref, args_mlp_w_ref,
                       args_mlp_b_ref, mlp_w_ref, mlp_b_ref, pos_w_ref,
                       tbl_ref, pos_ref):
    f32 = jnp.float32
    hi = jax.lax.Precision.HIGHEST
    mw_v = mlp_w_ref[0:4, :]
    mw_c = mlp_w_ref[4:12, :]
    mw_a = mlp_w_ref[12:140, :]
    cc = lax.broadcasted_iota(jnp.int32, (D_MODEL, D_MODEL), 1)
    src = ((cc // 32) * 32) + ((cc % 2) * 16) + ((cc % 32) // 2)
    perm = (lax.broadcasted_iota(jnp.int32, (D_MODEL, D_MODEL), 0)
            == src).astype(f32)
    vt = jnp.dot(view_w_ref[...], mw_v, precision=hi)
    ct = jnp.dot(cmd_w_ref[...], mw_c, precision=hi)
    r = lax.broadcasted_iota(jnp.int32, (VC_ROWS, 8), 0)
    ohv = (r // 7 == lax.broadcasted_iota(jnp.int32, (VC_ROWS, 8), 1)).astype(f32)
    ohc = (r % 7 == lax.broadcasted_iota(jnp.int32, (VC_ROWS, 8), 1)).astype(f32)
    vt8 = jnp.concatenate([vt, jnp.zeros((4, D_MODEL), f32)], axis=0)
    ct8 = jnp.concatenate([ct, jnp.zeros((1, D_MODEL), f32)], axis=0)
    vc = (jnp.dot(ohv, vt8, precision=hi)
          + jnp.dot(ohc, ct8, precision=hi))
    tbl_ref[0:VC_ROWS, :] = jnp.dot(vc, perm, precision=hi).astype(jnp.bfloat16)
    for j in range(N_ARGS):
        m_j = jnp.dot(args_mlp_w_ref[64 * j:64 * (j + 1), :], mw_a, precision=hi)
        at_j = jnp.dot(args_w_ref[...], m_j, precision=hi)
        base = VC_ROWS + j * TBL_STRIDE
        tbl_ref[base:base + 257, :] = jnp.dot(at_j, perm, precision=hi).astype(jnp.bfloat16)
        tbl_ref[base + 257:base + TBL_STRIDE, :] = jnp.zeros(
            (TBL_STRIDE - 257, D_MODEL), jnp.bfloat16)
    bias = mlp_b_ref[...] + jnp.dot(args_mlp_b_ref[...], mw_a, precision=hi)
    pos_ref[...] = pos_w_ref[0:S, :] + bias


def _build_tables(view_w, cmd_w, args_w, args_mlp_w, args_mlp_b, mlp_w, mlp_b, pos_w):
    return pl.pallas_call(
        _build_tables_body,
        out_shape=(
            jax.ShapeDtypeStruct((TBL_ROWS, D_MODEL), jnp.bfloat16),
            jax.ShapeDtypeStruct((S, D_MODEL), jnp.float32),
        ),
    )(view_w, cmd_w, args_w, args_mlp_w,
      args_mlp_b.reshape(1, -1), mlp_w, mlp_b.reshape(1, -1), pos_w)


def _sc_body(view_hbm, cmd_hbm, argst_hbm, tbl_hbm, pos_hbm, out_hbm,
             vrow, crow, arow, idx, posv, gbufA, gbufB, outvA, outvB,
             gsemA, gsemB, osemA, osemB):
    wid = lax.axis_index("s") * NC + lax.axis_index("c")

    def issue_gathers(ch, gbuf, gsem):
        base = ch * CHUNK
        for k in range(N_GATHER):
            pltpu.async_copy(
                tbl_hbm.at[idx.at[pl.ds(k * N + base, CHUNK)]],
                gbuf.at[pl.ds(k * CHUNK, CHUNK)], gsem)

    def wait_gathers(gbuf, gsem):
        pltpu.make_async_copy(
            tbl_hbm.at[pl.ds(0, N_GATHER * CHUNK)], gbuf, gsem).wait()

    def sum_chunk(s, ch, gbuf, outv):
        def do_tok(t, _):
            for wg in range(D_WORDS // 16):
                wsl = pl.ds(wg * 16, 16)
                acc_lo = posv[pl.ds(wg * 32, 16)]
                acc_hi = posv[pl.ds(wg * 32 + 16, 16)]
                for k in range(N_GATHER):
                    xw = gbuf[k * CHUNK + t, wsl]
                    acc_lo = acc_lo + plsc.bitcast(xw << 16, jnp.float32)
                    acc_hi = acc_hi + plsc.bitcast(
                        xw & jnp.int32(-65536), jnp.float32)
                outv[t, pl.ds(wg * 32, 16)] = acc_lo
                outv[t, pl.ds(wg * 32 + 16, 16)] = acc_hi
            return 0

        lax.fori_loop(0, CHUNK, do_tok, 0)

    def drain_out(outv, osem):
        pltpu.make_async_copy(outv, out_hbm.at[pl.ds(0, CHUNK)], osem).wait()

    def do_row(r, _):
        s = wid * ROWS_PER_W + r
        pltpu.sync_copy(view_hbm.at[pl.ds(s * N, N)], vrow)
        pltpu.sync_copy(cmd_hbm.at[pl.ds(s * N, N)], crow)
        for j in range(N_ARGS):
            pltpu.sync_copy(argst_hbm.at[pl.ds((j * S + s) * N, N)],
                            arow.at[pl.ds(j * N, N)])
        pltpu.sync_copy(pos_hbm.at[pl.ds(s * D_MODEL, D_MODEL)], posv)
        for g in range(N // 16):
            sl = pl.ds(g * 16, 16)
            idx[sl] = vrow[sl] * 7 + crow[sl]
            for j in range(N_ARGS):
                idx[pl.ds((1 + j) * N + g * 16, 16)] = (
                    arow[pl.ds(j * N + g * 16, 16)]
                    + (VC_ROWS + TBL_STRIDE * j + 1))

        issue_gathers(0, gbufA, gsemA)

        def do_pair(i, _):
            issue_gathers(2 * i + 1, gbufB, gsemB)
            wait_gathers(gbufA, gsemA)

            @pl.when(i > 0)
            def _():
                drain_out(outvA, osemA)

            sum_chunk(s, 2 * i, gbufA, outvA)
            pltpu.async_copy(
                outvA, out_hbm.at[pl.ds(s * N + 2 * i * CHUNK, CHUNK)], osemA)

            @pl.when(i < (N // CHUNK) // 2 - 1)
            def _():
                issue_gathers(2 * i + 2, gbufA, gsemA)

            wait_gathers(gbufB, gsemB)

            @pl.when(i > 0)
            def _():
                drain_out(outvB, osemB)

            sum_chunk(s, 2 * i + 1, gbufB, outvB)
            pltpu.async_copy(
                outvB, out_hbm.at[pl.ds(s * N + (2 * i + 1) * CHUNK, CHUNK)],
                osemB)
            return 0

        lax.fori_loop(0, (N // CHUNK) // 2, do_pair, 0)
        drain_out(outvA, osemA)
        drain_out(outvB, osemB)
        return 0

    lax.fori_loop(0, ROWS_PER_W, do_row, 0)


def _sc_bag(view, cmd, argst, tbl, pos):
    mesh = plsc.VectorSubcoreMesh(core_axis_name="c", subcore_axis_name="s")
    return pl.kernel(
        _sc_body,
        out_type=jax.ShapeDtypeStruct((S * N, D_MODEL), jnp.float32),
        mesh=mesh,
        compiler_params=pltpu.CompilerParams(needs_layout_passes=False),
        scratch_types=[
            pltpu.VMEM((N,), jnp.int32),
            pltpu.VMEM((N,), jnp.int32),
            pltpu.VMEM((N_ARGS * N,), jnp.int32),
            pltpu.VMEM((N_GATHER * N,), jnp.int32),
            pltpu.VMEM((D_MODEL,), jnp.float32),
            pltpu.VMEM((N_GATHER * CHUNK, D_WORDS), jnp.int32),
            pltpu.VMEM((N_GATHER * CHUNK, D_WORDS), jnp.int32),
            pltpu.VMEM((CHUNK, D_MODEL), jnp.float32),
            pltpu.VMEM((CHUNK, D_MODEL), jnp.float32),
            pltpu.SemaphoreType.DMA,
            pltpu.SemaphoreType.DMA,
            pltpu.SemaphoreType.DMA,
            pltpu.SemaphoreType.DMA,
        ],
    )(view, cmd, argst, tbl, pos)


def kernel(view, command, args, view_embed_w, command_embed_w, args_embed_w,
           args_mlp_w, args_mlp_b, mlp_w, mlp_b, pos_embed_w):
    tbl, pos = _build_tables(view_embed_w, command_embed_w, args_embed_w,
                             args_mlp_w, args_mlp_b, mlp_w, mlp_b, pos_embed_w)
    argst = args.transpose(2, 0, 1).reshape(-1)
    tbl_i32 = lax.bitcast_convert_type(
        tbl.reshape(TBL_ROWS, D_WORDS, 2), jnp.int32)
    out = _sc_bag(view.reshape(-1), command.reshape(-1), argst, tbl_i32,
                  pos.reshape(-1))
    return out.reshape(S, N, D_MODEL)


# speedup vs baseline: 10.2930x; 1.0733x over previous
import functools

import jax
import jax.numpy as jnp
from jax import lax
from jax.experimental import pallas as pl
from jax.experimental.pallas import tpu as pltpu
from jax.experimental.pallas import tpu_sc as plsc

D_MODEL = 256
N_ARGS = 11
S, N = 512, 512
VC_ROWS = 32
TBL_STRIDE = 264
TBL_ROWS = VC_ROWS + N_ARGS * TBL_STRIDE
NC, NS = 2, 16
NW = NC * NS
ROWS_PER_W = S // NW
CHUNK = 16
N_GATHER = 12
D_WORDS = D_MODEL // 2


def _build_tables_body(view_w_ref, cmd_w_ref, args_w_ref, args_mlp_w_ref,
                       args_mlp_b_ref, mlp_w_ref, mlp_b_ref, pos_w_ref,
                       tbl_ref, pos_ref):
    f32 = jnp.float32
    hi = jax.lax.Precision.HIGHEST
    mw_v = mlp_w_ref[0:4, :]
    mw_c = mlp_w_ref[4:12, :]
    mw_a = mlp_w_ref[12:140, :]
    cc = lax.broadcasted_iota(jnp.int32, (D_MODEL, D_MODEL), 1)
    src = ((cc // 32) * 32) + ((cc % 2) * 16) + ((cc % 32) // 2)
    perm = (lax.broadcasted_iota(jnp.int32, (D_MODEL, D_MODEL), 0)
            == src).astype(f32)
    vt = jnp.dot(view_w_ref[...], mw_v, precision=hi)
    ct = jnp.dot(cmd_w_ref[...], mw_c, precision=hi)
    r = lax.broadcasted_iota(jnp.int32, (VC_ROWS, 8), 0)
    ohv = (r // 7 == lax.broadcasted_iota(jnp.int32, (VC_ROWS, 8), 1)).astype(f32)
    ohc = (r % 7 == lax.broadcasted_iota(jnp.int32, (VC_ROWS, 8), 1)).astype(f32)
    vt8 = jnp.concatenate([vt, jnp.zeros((4, D_MODEL), f32)], axis=0)
    ct8 = jnp.concatenate([ct, jnp.zeros((1, D_MODEL), f32)], axis=0)
    vc = (jnp.dot(ohv, vt8, precision=hi)
          + jnp.dot(ohc, ct8, precision=hi))
    tbl_ref[0:VC_ROWS, :] = jnp.dot(vc, perm, precision=hi).astype(jnp.bfloat16)
    for j in range(N_ARGS):
        m_j = jnp.dot(args_mlp_w_ref[64 * j:64 * (j + 1), :], mw_a, precision=hi)
        at_j = jnp.dot(args_w_ref[...], m_j, precision=hi)
        base = VC_ROWS + j * TBL_STRIDE
        tbl_ref[base:base + 257, :] = jnp.dot(at_j, perm, precision=hi).astype(jnp.bfloat16)
        tbl_ref[base + 257:base + TBL_STRIDE, :] = jnp.zeros(
            (TBL_STRIDE - 257, D_MODEL), jnp.bfloat16)
    bias = mlp_b_ref[...] + jnp.dot(args_mlp_b_ref[...], mw_a, precision=hi)
    pos_ref[...] = pos_w_ref[0:S, :] + bias


def _build_tables(view_w, cmd_w, args_w, args_mlp_w, args_mlp_b, mlp_w, mlp_b, pos_w):
    return pl.pallas_call(
        _build_tables_body,
        out_shape=(
            jax.ShapeDtypeStruct((TBL_ROWS, D_MODEL), jnp.bfloat16),
            jax.ShapeDtypeStruct((S, D_MODEL), jnp.float32),
        ),
    )(view_w, cmd_w, args_w, args_mlp_w,
      args_mlp_b.reshape(1, -1), mlp_w, mlp_b.reshape(1, -1), pos_w)


def _sc_body(view_hbm, cmd_hbm, argst_hbm, tbl_hbm, pos_hbm, out_hbm,
             vrow, crow, arow, idx, posv, gbufA, gbufB, outvA, outvB, tbl_sh,
             gsemA, gsemB, osemA, osemB):
    wid = lax.axis_index("s") * NC + lax.axis_index("c")

    @pl.when(lax.axis_index("s") == 0)
    def _():
        pltpu.sync_copy(tbl_hbm, tbl_sh)

    plsc.subcore_barrier()

    def issue_gathers(ch, gbuf, gsem):
        base = ch * CHUNK
        for k in range(N_GATHER):
            pltpu.async_copy(
                tbl_sh.at[idx.at[pl.ds(k * N + base, CHUNK)]],
                gbuf.at[pl.ds(k * CHUNK, CHUNK)], gsem)

    def wait_gathers(gbuf, gsem):
        pltpu.make_async_copy(
            tbl_hbm.at[pl.ds(0, N_GATHER * CHUNK)], gbuf, gsem).wait()

    def sum_chunk(s, ch, gbuf, outv):
        def do_tok(t, _):
            for wg in range(D_WORDS // 16):
                wsl = pl.ds(wg * 16, 16)
                acc_lo = posv[pl.ds(wg * 32, 16)]
                acc_hi = posv[pl.ds(wg * 32 + 16, 16)]
                for k in range(N_GATHER):
                    xw = gbuf[k * CHUNK + t, wsl]
                    acc_lo = acc_lo + plsc.bitcast(xw << 16, jnp.float32)
                    acc_hi = acc_hi + plsc.bitcast(
                        xw & jnp.int32(-65536), jnp.float32)
                outv[t, pl.ds(wg * 32, 16)] = acc_lo
                outv[t, pl.ds(wg * 32 + 16, 16)] = acc_hi
            return 0

        lax.fori_loop(0, CHUNK, do_tok, 0)

    def drain_out(outv, osem):
        pltpu.make_async_copy(outv, out_hbm.at[pl.ds(0, CHUNK)], osem).wait()

    def do_row(r, _):
        s = wid * ROWS_PER_W + r
        pltpu.sync_copy(view_hbm.at[pl.ds(s * N, N)], vrow)
        pltpu.sync_copy(cmd_hbm.at[pl.ds(s * N, N)], crow)
        for j in range(N_ARGS):
            pltpu.sync_copy(argst_hbm.at[pl.ds((j * S + s) * N, N)],
                            arow.at[pl.ds(j * N, N)])
        pltpu.sync_copy(pos_hbm.at[pl.ds(s * D_MODEL, D_MODEL)], posv)
        for g in range(N // 16):
            sl = pl.ds(g * 16, 16)
            idx[sl] = vrow[sl] * 7 + crow[sl]
            for j in range(N_ARGS):
                idx[pl.ds((1 + j) * N + g * 16, 16)] = (
                    arow[pl.ds(j * N + g * 16, 16)]
                    + (VC_ROWS + TBL_STRIDE * j + 1))

        issue_gathers(0, gbufA, gsemA)

        def do_pair(i, _):
            issue_gathers(2 * i + 1, gbufB, gsemB)
            wait_gathers(gbufA, gsemA)

            @pl.when(i > 0)
            def _():
                drain_out(outvA, osemA)

            sum_chunk(s, 2 * i, gbufA, outvA)
            pltpu.async_copy(
                outvA, out_hbm.at[pl.ds(s * N + 2 * i * CHUNK, CHUNK)], osemA)

            @pl.when(i < (N // CHUNK) // 2 - 1)
            def _():
                issue_gathers(2 * i + 2, gbufA, gsemA)

            wait_gathers(gbufB, gsemB)

            @pl.when(i > 0)
            def _():
                drain_out(outvB, osemB)

            sum_chunk(s, 2 * i + 1, gbufB, outvB)
            pltpu.async_copy(
                outvB, out_hbm.at[pl.ds(s * N + (2 * i + 1) * CHUNK, CHUNK)],
                osemB)
            return 0

        lax.fori_loop(0, (N // CHUNK) // 2, do_pair, 0)
        drain_out(outvA, osemA)
        drain_out(outvB, osemB)
        return 0

    lax.fori_loop(0, ROWS_PER_W, do_row, 0)


def _sc_bag(view, cmd, argst, tbl, pos):
    mesh = plsc.VectorSubcoreMesh(core_axis_name="c", subcore_axis_name="s")
    return pl.kernel(
        _sc_body,
        out_type=jax.ShapeDtypeStruct((S * N, D_MODEL), jnp.float32),
        mesh=mesh,
        compiler_params=pltpu.CompilerParams(needs_layout_passes=False),
        scratch_types=[
            pltpu.VMEM((N,), jnp.int32),
            pltpu.VMEM((N,), jnp.int32),
            pltpu.VMEM((N_ARGS * N,), jnp.int32),
            pltpu.VMEM((N_GATHER * N,), jnp.int32),
            pltpu.VMEM((D_MODEL,), jnp.float32),
            pltpu.VMEM((N_GATHER * CHUNK, D_WORDS), jnp.int32),
            pltpu.VMEM((N_GATHER * CHUNK, D_WORDS), jnp.int32),
            pltpu.VMEM((CHUNK, D_MODEL), jnp.float32),
            pltpu.VMEM((CHUNK, D_MODEL), jnp.float32),
            pltpu.VMEM_SHARED((TBL_ROWS, D_WORDS), jnp.int32),
            pltpu.SemaphoreType.DMA,
            pltpu.SemaphoreType.DMA,
            pltpu.SemaphoreType.DMA,
            pltpu.SemaphoreType.DMA,
        ],
    )(view, cmd, argst, tbl, pos)


def kernel(view, command, args, view_embed_w, command_embed_w, args_embed_w,
           args_mlp_w, args_mlp_b, mlp_w, mlp_b, pos_embed_w):
    tbl, pos = _build_tables(view_embed_w, command_embed_w, args_embed_w,
                             args_mlp_w, args_mlp_b, mlp_w, mlp_b, pos_embed_w)
    argst = args.transpose(2, 0, 1).reshape(-1)
    tbl_i32 = lax.bitcast_convert_type(
        tbl.reshape(TBL_ROWS, D_WORDS, 2), jnp.int32)
    out = _sc_bag(view.reshape(-1), command.reshape(-1), argst, tbl_i32,
                  pos.reshape(-1))
    return out.reshape(S, N, D_MODEL)


# speedup vs baseline: 11.7494x; 1.1415x over previous
import functools

import jax
import jax.numpy as jnp
from jax import lax
from jax.experimental import pallas as pl
from jax.experimental.pallas import tpu as pltpu
from jax.experimental.pallas import tpu_sc as plsc

D_MODEL = 256
N_ARGS = 11
S, N = 512, 512
VC_ROWS = 32
TBL_STRIDE = 264
TBL_ROWS = VC_ROWS + N_ARGS * TBL_STRIDE
NC, NS = 2, 16
NW = NC * NS
ROWS_PER_W = S // NW
CHUNK = 16
N_GATHER = 12
D_WORDS = D_MODEL // 2


def _build_tables_body(view_w_ref, cmd_w_ref, args_w_ref, args_mlp_w_ref,
                       args_mlp_b_ref, mlp_w_ref, mlp_b_ref, pos_w_ref,
                       tbl_ref, pos_ref):
    f32 = jnp.float32
    hi = jax.lax.Precision.HIGHEST
    mw_v = mlp_w_ref[0:4, :]
    mw_c = mlp_w_ref[4:12, :]
    mw_a = mlp_w_ref[12:140, :]
    cc = lax.broadcasted_iota(jnp.int32, (D_MODEL, D_MODEL), 1)
    src = ((cc // 32) * 32) + ((cc % 2) * 16) + ((cc % 32) // 2)
    perm = (lax.broadcasted_iota(jnp.int32, (D_MODEL, D_MODEL), 0)
            == src).astype(f32)
    vt = jnp.dot(view_w_ref[...], mw_v, precision=hi)
    ct = jnp.dot(cmd_w_ref[...], mw_c, precision=hi)
    r = lax.broadcasted_iota(jnp.int32, (VC_ROWS, 8), 0)
    ohv = (r // 7 == lax.broadcasted_iota(jnp.int32, (VC_ROWS, 8), 1)).astype(f32)
    ohc = (r % 7 == lax.broadcasted_iota(jnp.int32, (VC_ROWS, 8), 1)).astype(f32)
    vt8 = jnp.concatenate([vt, jnp.zeros((4, D_MODEL), f32)], axis=0)
    ct8 = jnp.concatenate([ct, jnp.zeros((1, D_MODEL), f32)], axis=0)
    vc = (jnp.dot(ohv, vt8, precision=hi)
          + jnp.dot(ohc, ct8, precision=hi))
    tbl_ref[0:VC_ROWS, :] = jnp.dot(vc, perm, precision=hi).astype(jnp.bfloat16)
    for j in range(N_ARGS):
        m_j = jnp.dot(args_mlp_w_ref[64 * j:64 * (j + 1), :], mw_a, precision=hi)
        at_j = jnp.dot(args_w_ref[...], m_j, precision=hi)
        base = VC_ROWS + j * TBL_STRIDE
        tbl_ref[base:base + 257, :] = jnp.dot(at_j, perm, precision=hi).astype(jnp.bfloat16)
        tbl_ref[base + 257:base + TBL_STRIDE, :] = jnp.zeros(
            (TBL_STRIDE - 257, D_MODEL), jnp.bfloat16)
    bias = mlp_b_ref[...] + jnp.dot(args_mlp_b_ref[...], mw_a, precision=hi)
    pos_ref[...] = pos_w_ref[0:S, :] + bias


def _build_tables(view_w, cmd_w, args_w, args_mlp_w, args_mlp_b, mlp_w, mlp_b, pos_w):
    return pl.pallas_call(
        _build_tables_body,
        out_shape=(
            jax.ShapeDtypeStruct((TBL_ROWS, D_MODEL), jnp.bfloat16),
            jax.ShapeDtypeStruct((S, D_MODEL), jnp.float32),
        ),
    )(view_w, cmd_w, args_w, args_mlp_w,
      args_mlp_b.reshape(1, -1), mlp_w, mlp_b.reshape(1, -1), pos_w)


def _sc_body(view_hbm, cmd_hbm, argst_hbm, tbl_hbm, pos_hbm, out_hbm,
             vrow, crow, arow, idx, posv, gbufA, gbufB, outvA, outvB, tbl_sh,
             gsemA, gsemB, osemA, osemB):
    wid = lax.axis_index("s") * NC + lax.axis_index("c")

    @pl.when(lax.axis_index("s") == 0)
    def _():
        pltpu.sync_copy(tbl_hbm, tbl_sh)

    plsc.subcore_barrier()

    def issue_gathers(ch, gbuf, gsem):
        base = ch * CHUNK
        for k in range(N_GATHER):
            pltpu.async_copy(
                tbl_sh.at[idx.at[pl.ds(k * N + base, CHUNK)]],
                gbuf.at[pl.ds(k * CHUNK, CHUNK)], gsem)

    def wait_gathers(gbuf, gsem):
        pltpu.make_async_copy(
            tbl_hbm.at[pl.ds(0, N_GATHER * CHUNK)], gbuf, gsem).wait()

    def sum_chunk(s, ch, gbuf, outv):
        pos_regs = [posv[pl.ds(i * 16, 16)] for i in range(D_MODEL // 16)]

        def do_tok(t, _):
            for wg in range(D_WORDS // 16):
                wsl = pl.ds(wg * 16, 16)
                bs = [plsc.bitcast(gbuf[k * CHUNK + t, wsl], jnp.bfloat16)
                      for k in range(N_GATHER)]
                while len(bs) > 1:
                    bs = [bs[i] + bs[i + 1] for i in range(0, len(bs) - 1, 2)]                         + ([bs[-1]] if len(bs) % 2 else [])
                xw = plsc.bitcast(bs[0], jnp.int32)
                lo = plsc.bitcast(xw << 16, jnp.float32)
                hi = plsc.bitcast(xw & jnp.int32(-65536), jnp.float32)
                outv[t, pl.ds(wg * 32, 16)] = lo + pos_regs[2 * wg]
                outv[t, pl.ds(wg * 32 + 16, 16)] = hi + pos_regs[2 * wg + 1]
            return 0

        lax.fori_loop(0, CHUNK, do_tok, 0)

    def drain_out(outv, osem):
        pltpu.make_async_copy(outv, out_hbm.at[pl.ds(0, CHUNK)], osem).wait()

    def do_row(r, _):
        s = wid * ROWS_PER_W + r
        pltpu.sync_copy(view_hbm.at[pl.ds(s * N, N)], vrow)
        pltpu.sync_copy(cmd_hbm.at[pl.ds(s * N, N)], crow)
        for j in range(N_ARGS):
            pltpu.sync_copy(argst_hbm.at[pl.ds((j * S + s) * N, N)],
                            arow.at[pl.ds(j * N, N)])
        pltpu.sync_copy(pos_hbm.at[pl.ds(s * D_MODEL, D_MODEL)], posv)
        for g in range(N // 16):
            sl = pl.ds(g * 16, 16)
            idx[sl] = vrow[sl] * 7 + crow[sl]
            for j in range(N_ARGS):
                idx[pl.ds((1 + j) * N + g * 16, 16)] = (
                    arow[pl.ds(j * N + g * 16, 16)]
                    + (VC_ROWS + TBL_STRIDE * j + 1))

        issue_gathers(0, gbufA, gsemA)

        def do_pair(i, _):
            issue_gathers(2 * i + 1, gbufB, gsemB)
            wait_gathers(gbufA, gsemA)

            @pl.when(i > 0)
            def _():
                drain_out(outvA, osemA)

            sum_chunk(s, 2 * i, gbufA, outvA)
            pltpu.async_copy(
                outvA, out_hbm.at[pl.ds(s * N + 2 * i * CHUNK, CHUNK)], osemA)

            @pl.when(i < (N // CHUNK) // 2 - 1)
            def _():
                issue_gathers(2 * i + 2, gbufA, gsemA)

            wait_gathers(gbufB, gsemB)

            @pl.when(i > 0)
            def _():
                drain_out(outvB, osemB)

            sum_chunk(s, 2 * i + 1, gbufB, outvB)
            pltpu.async_copy(
                outvB, out_hbm.at[pl.ds(s * N + (2 * i + 1) * CHUNK, CHUNK)],
                osemB)
            return 0

        lax.fori_loop(0, (N // CHUNK) // 2, do_pair, 0)
        drain_out(outvA, osemA)
        drain_out(outvB, osemB)
        return 0

    lax.fori_loop(0, ROWS_PER_W, do_row, 0)


def _sc_bag(view, cmd, argst, tbl, pos):
    mesh = plsc.VectorSubcoreMesh(core_axis_name="c", subcore_axis_name="s")
    return pl.kernel(
        _sc_body,
        out_type=jax.ShapeDtypeStruct((S * N, D_MODEL), jnp.float32),
        mesh=mesh,
        compiler_params=pltpu.CompilerParams(needs_layout_passes=False),
        scratch_types=[
            pltpu.VMEM((N,), jnp.int32),
            pltpu.VMEM((N,), jnp.int32),
            pltpu.VMEM((N_ARGS * N,), jnp.int32),
            pltpu.VMEM((N_GATHER * N,), jnp.int32),
            pltpu.VMEM((D_MODEL,), jnp.float32),
            pltpu.VMEM((N_GATHER * CHUNK, D_WORDS), jnp.int32),
            pltpu.VMEM((N_GATHER * CHUNK, D_WORDS), jnp.int32),
            pltpu.VMEM((CHUNK, D_MODEL), jnp.float32),
            pltpu.VMEM((CHUNK, D_MODEL), jnp.float32),
            pltpu.VMEM_SHARED((TBL_ROWS, D_WORDS), jnp.int32),
            pltpu.SemaphoreType.DMA,
            pltpu.SemaphoreType.DMA,
            pltpu.SemaphoreType.DMA,
            pltpu.SemaphoreType.DMA,
        ],
    )(view, cmd, argst, tbl, pos)


def kernel(view, command, args, view_embed_w, command_embed_w, args_embed_w,
           args_mlp_w, args_mlp_b, mlp_w, mlp_b, pos_embed_w):
    tbl, pos = _build_tables(view_embed_w, command_embed_w, args_embed_w,
                             args_mlp_w, args_mlp_b, mlp_w, mlp_b, pos_embed_w)
    argst = args.transpose(2, 0, 1).reshape(-1)
    tbl_i32 = lax.bitcast_convert_type(
        tbl.reshape(TBL_ROWS, D_WORDS, 2), jnp.int32)
    out = _sc_bag(view.reshape(-1), command.reshape(-1), argst, tbl_i32,
                  pos.reshape(-1))
    return out.reshape(S, N, D_MODEL)


# speedup vs baseline: 15.4622x; 1.3160x over previous
import functools

import jax
import jax.numpy as jnp
from jax import lax
from jax.experimental import pallas as pl
from jax.experimental.pallas import tpu as pltpu
from jax.experimental.pallas import tpu_sc as plsc

D_MODEL = 256
N_ARGS = 11
S, N = 512, 512
VC_ROWS = 32
TBL_STRIDE = 264
TBL_ROWS = VC_ROWS + N_ARGS * TBL_STRIDE
NC, NS = 2, 16
NW = NC * NS
ROWS_PER_W = S // NW
CHUNK = 16
N_GATHER = 12
D_WORDS = D_MODEL // 2


def _build_tables_body(view_w_ref, cmd_w_ref, args_w_ref, args_mlp_w_ref,
                       args_mlp_b_ref, mlp_w_ref, mlp_b_ref, pos_w_ref,
                       tbl_ref, pos_ref):
    f32 = jnp.float32
    hi = jax.lax.Precision.HIGHEST
    mw_v = mlp_w_ref[0:4, :]
    mw_c = mlp_w_ref[4:12, :]
    mw_a = mlp_w_ref[12:140, :]
    cc = lax.broadcasted_iota(jnp.int32, (D_MODEL, D_MODEL), 1)
    src = ((cc // 32) * 32) + ((cc % 2) * 16) + ((cc % 32) // 2)
    perm = (lax.broadcasted_iota(jnp.int32, (D_MODEL, D_MODEL), 0)
            == src).astype(f32)
    vt = jnp.dot(view_w_ref[...], mw_v, precision=hi)
    ct = jnp.dot(cmd_w_ref[...], mw_c, precision=hi)
    r = lax.broadcasted_iota(jnp.int32, (VC_ROWS, 8), 0)
    ohv = (r // 7 == lax.broadcasted_iota(jnp.int32, (VC_ROWS, 8), 1)).astype(f32)
    ohc = (r % 7 == lax.broadcasted_iota(jnp.int32, (VC_ROWS, 8), 1)).astype(f32)
    vt8 = jnp.concatenate([vt, jnp.zeros((4, D_MODEL), f32)], axis=0)
    ct8 = jnp.concatenate([ct, jnp.zeros((1, D_MODEL), f32)], axis=0)
    vc = (jnp.dot(ohv, vt8, precision=hi)
          + jnp.dot(ohc, ct8, precision=hi))
    tbl_ref[0:VC_ROWS, :] = jnp.dot(vc, perm, precision=hi).astype(jnp.bfloat16)
    for j in range(N_ARGS):
        m_j = jnp.dot(args_mlp_w_ref[64 * j:64 * (j + 1), :], mw_a, precision=hi)
        at_j = jnp.dot(args_w_ref[...], m_j, precision=hi)
        base = VC_ROWS + j * TBL_STRIDE
        tbl_ref[base:base + 257, :] = jnp.dot(at_j, perm, precision=hi).astype(jnp.bfloat16)
        tbl_ref[base + 257:base + TBL_STRIDE, :] = jnp.zeros(
            (TBL_STRIDE - 257, D_MODEL), jnp.bfloat16)
    bias = mlp_b_ref[...] + jnp.dot(args_mlp_b_ref[...], mw_a, precision=hi)
    pos_ref[...] = pos_w_ref[0:S, :] + bias


def _build_tables(view_w, cmd_w, args_w, args_mlp_w, args_mlp_b, mlp_w, mlp_b, pos_w):
    return pl.pallas_call(
        _build_tables_body,
        out_shape=(
            jax.ShapeDtypeStruct((TBL_ROWS, D_MODEL), jnp.bfloat16),
            jax.ShapeDtypeStruct((S, D_MODEL), jnp.float32),
        ),
    )(view_w, cmd_w, args_w, args_mlp_w,
      args_mlp_b.reshape(1, -1), mlp_w, mlp_b.reshape(1, -1), pos_w)


def _sc_body(view_hbm, cmd_hbm, argst_hbm, tbl_hbm, pos_hbm, out_hbm,
             vrow, crow, arow, idx, posv, gbufA, gbufB, outvA, outvB, tbl_sh,
             gsemA, gsemB, osemA, osemB):
    wid = lax.axis_index("s") * NC + lax.axis_index("c")

    @pl.when(lax.axis_index("s") == 0)
    def _():
        pltpu.sync_copy(tbl_hbm, tbl_sh)

    plsc.subcore_barrier()

    def issue_gathers(ch, gbuf, gsem):
        base = ch * CHUNK
        for k in range(N_GATHER):
            pltpu.async_copy(
                tbl_sh.at[idx.at[pl.ds(k * N + base, CHUNK)]],
                gbuf.at[pl.ds(k * CHUNK, CHUNK)], gsem)

    def wait_gathers(gbuf, gsem):
        pltpu.make_async_copy(
            tbl_hbm.at[pl.ds(0, N_GATHER * CHUNK)], gbuf, gsem).wait()

    def sum_chunk(s, ch, gbuf, outv):
        pos_regs = [posv[pl.ds(i * 16, 16)] for i in range(D_MODEL // 16)]

        def tree(bs):
            while len(bs) > 1:
                bs = ([bs[i] + bs[i + 1] for i in range(0, len(bs) - 1, 2)]
                      + ([bs[-1]] if len(bs) % 2 else []))
            return bs[0]

        def do_tok(t, _):
            for wg in range(0, D_WORDS // 16, 2):
                bsA = [plsc.bitcast(gbuf[k * CHUNK + t, pl.ds(wg * 16, 16)],
                                    jnp.bfloat16) for k in range(N_GATHER)]
                bsB = [plsc.bitcast(gbuf[k * CHUNK + t, pl.ds(wg * 16 + 16, 16)],
                                    jnp.bfloat16) for k in range(N_GATHER)]
                for h, bs in ((wg, bsA), (wg + 1, bsB)):
                    xw = plsc.bitcast(tree(bs), jnp.int32)
                    lo = plsc.bitcast(xw << 16, jnp.float32)
                    hi = plsc.bitcast(xw & jnp.int32(-65536), jnp.float32)
                    outv[t, pl.ds(h * 32, 16)] = lo + pos_regs[2 * h]
                    outv[t, pl.ds(h * 32 + 16, 16)] = hi + pos_regs[2 * h + 1]
            return 0

        lax.fori_loop(0, CHUNK, do_tok, 0)

    def drain_out(outv, osem):
        pltpu.make_async_copy(outv, out_hbm.at[pl.ds(0, CHUNK)], osem).wait()

    def do_row(r, _):
        s = wid * ROWS_PER_W + r
        pltpu.sync_copy(view_hbm.at[pl.ds(s * N, N)], vrow)
        pltpu.sync_copy(cmd_hbm.at[pl.ds(s * N, N)], crow)
        for j in range(N_ARGS):
            pltpu.sync_copy(argst_hbm.at[pl.ds((j * S + s) * N, N)],
                            arow.at[pl.ds(j * N, N)])
        pltpu.sync_copy(pos_hbm.at[pl.ds(s * D_MODEL, D_MODEL)], posv)
        for g in range(N // 16):
            sl = pl.ds(g * 16, 16)
            idx[sl] = vrow[sl] * 7 + crow[sl]
            for j in range(N_ARGS):
                idx[pl.ds((1 + j) * N + g * 16, 16)] = (
                    arow[pl.ds(j * N + g * 16, 16)]
                    + (VC_ROWS + TBL_STRIDE * j + 1))

        issue_gathers(0, gbufA, gsemA)

        def do_pair(i, _):
            issue_gathers(2 * i + 1, gbufB, gsemB)
            wait_gathers(gbufA, gsemA)

            @pl.when(i > 0)
            def _():
                drain_out(outvA, osemA)

            sum_chunk(s, 2 * i, gbufA, outvA)
            pltpu.async_copy(
                outvA, out_hbm.at[pl.ds(s * N + 2 * i * CHUNK, CHUNK)], osemA)

            @pl.when(i < (N // CHUNK) // 2 - 1)
            def _():
                issue_gathers(2 * i + 2, gbufA, gsemA)

            wait_gathers(gbufB, gsemB)

            @pl.when(i > 0)
            def _():
                drain_out(outvB, osemB)

            sum_chunk(s, 2 * i + 1, gbufB, outvB)
            pltpu.async_copy(
                outvB, out_hbm.at[pl.ds(s * N + (2 * i + 1) * CHUNK, CHUNK)],
                osemB)
            return 0

        lax.fori_loop(0, (N // CHUNK) // 2, do_pair, 0)
        drain_out(outvA, osemA)
        drain_out(outvB, osemB)
        return 0

    lax.fori_loop(0, ROWS_PER_W, do_row, 0)


def _sc_bag(view, cmd, argst, tbl, pos):
    mesh = plsc.VectorSubcoreMesh(core_axis_name="c", subcore_axis_name="s")
    return pl.kernel(
        _sc_body,
        out_type=jax.ShapeDtypeStruct((S * N, D_MODEL), jnp.float32),
        mesh=mesh,
        compiler_params=pltpu.CompilerParams(needs_layout_passes=False),
        scratch_types=[
            pltpu.VMEM((N,), jnp.int32),
            pltpu.VMEM((N,), jnp.int32),
            pltpu.VMEM((N_ARGS * N,), jnp.int32),
            pltpu.VMEM((N_GATHER * N,), jnp.int32),
            pltpu.VMEM((D_MODEL,), jnp.float32),
            pltpu.VMEM((N_GATHER * CHUNK, D_WORDS), jnp.int32),
            pltpu.VMEM((N_GATHER * CHUNK, D_WORDS), jnp.int32),
            pltpu.VMEM((CHUNK, D_MODEL), jnp.float32),
            pltpu.VMEM((CHUNK, D_MODEL), jnp.float32),
            pltpu.VMEM_SHARED((TBL_ROWS, D_WORDS), jnp.int32),
            pltpu.SemaphoreType.DMA,
            pltpu.SemaphoreType.DMA,
            pltpu.SemaphoreType.DMA,
            pltpu.SemaphoreType.DMA,
        ],
    )(view, cmd, argst, tbl, pos)


def kernel(view, command, args, view_embed_w, command_embed_w, args_embed_w,
           args_mlp_w, args_mlp_b, mlp_w, mlp_b, pos_embed_w):
    tbl, pos = _build_tables(view_embed_w, command_embed_w, args_embed_w,
                             args_mlp_w, args_mlp_b, mlp_w, mlp_b, pos_embed_w)
    argst = args.transpose(2, 0, 1).reshape(-1)
    tbl_i32 = lax.bitcast_convert_type(
        tbl.reshape(TBL_ROWS, D_WORDS, 2), jnp.int32)
    out = _sc_bag(view.reshape(-1), command.reshape(-1), argst, tbl_i32,
                  pos.reshape(-1))
    return out.reshape(S, N, D_MODEL)


# speedup vs baseline: 16.7953x; 1.0862x over previous
import functools

import jax
import jax.numpy as jnp
from jax import lax
from jax.experimental import pallas as pl
from jax.experimental.pallas import tpu as pltpu
from jax.experimental.pallas import tpu_sc as plsc

D_MODEL = 256
N_ARGS = 11
S, N = 512, 512
VC_ROWS = 32
TBL_STRIDE = 264
TBL_ROWS = VC_ROWS + N_ARGS * TBL_STRIDE
NC, NS = 2, 16
NW = NC * NS
ROWS_PER_W = S // NW
CHUNK = 16
N_GATHER = 12
D_WORDS = D_MODEL // 2


def _build_tables_body(view_w_ref, cmd_w_ref, args_w_ref, args_mlp_w_ref,
                       args_mlp_b_ref, mlp_w_ref, mlp_b_ref, pos_w_ref,
                       tbl_ref, pos_ref):
    f32 = jnp.float32
    hi = jax.lax.Precision.HIGHEST
    mw_v = mlp_w_ref[0:4, :]
    mw_c = mlp_w_ref[4:12, :]
    mw_a = mlp_w_ref[12:140, :]
    cc = lax.broadcasted_iota(jnp.int32, (D_MODEL, D_MODEL), 1)
    src = ((cc // 32) * 32) + ((cc % 2) * 16) + ((cc % 32) // 2)
    perm = (lax.broadcasted_iota(jnp.int32, (D_MODEL, D_MODEL), 0)
            == src).astype(f32)
    vt = jnp.dot(view_w_ref[...], mw_v, precision=hi)
    ct = jnp.dot(cmd_w_ref[...], mw_c, precision=hi)
    r = lax.broadcasted_iota(jnp.int32, (VC_ROWS, 8), 0)
    ohv = (r // 7 == lax.broadcasted_iota(jnp.int32, (VC_ROWS, 8), 1)).astype(f32)
    ohc = (r % 7 == lax.broadcasted_iota(jnp.int32, (VC_ROWS, 8), 1)).astype(f32)
    vt8 = jnp.concatenate([vt, jnp.zeros((4, D_MODEL), f32)], axis=0)
    ct8 = jnp.concatenate([ct, jnp.zeros((1, D_MODEL), f32)], axis=0)
    vc = (jnp.dot(ohv, vt8, precision=hi)
          + jnp.dot(ohc, ct8, precision=hi))
    tbl_ref[0:VC_ROWS, :] = jnp.dot(vc, perm, precision=hi).astype(jnp.bfloat16)
    for j in range(N_ARGS):
        m_j = jnp.dot(args_mlp_w_ref[64 * j:64 * (j + 1), :], mw_a, precision=hi)
        at_j = jnp.dot(args_w_ref[...], m_j, precision=hi)
        base = VC_ROWS + j * TBL_STRIDE
        tbl_ref[base:base + 257, :] = jnp.dot(at_j, perm, precision=hi).astype(jnp.bfloat16)
        tbl_ref[base + 257:base + TBL_STRIDE, :] = jnp.zeros(
            (TBL_STRIDE - 257, D_MODEL), jnp.bfloat16)
    bias = mlp_b_ref[...] + jnp.dot(args_mlp_b_ref[...], mw_a, precision=hi)
    pos_ref[...] = pos_w_ref[0:S, :] + bias


def _build_tables(view_w, cmd_w, args_w, args_mlp_w, args_mlp_b, mlp_w, mlp_b, pos_w):
    return pl.pallas_call(
        _build_tables_body,
        out_shape=(
            jax.ShapeDtypeStruct((TBL_ROWS, D_MODEL), jnp.bfloat16),
            jax.ShapeDtypeStruct((S, D_MODEL), jnp.float32),
        ),
    )(view_w, cmd_w, args_w, args_mlp_w,
      args_mlp_b.reshape(1, -1), mlp_w, mlp_b.reshape(1, -1), pos_w)


CAT_L = 13 * N + D_MODEL


def _sc_body(cat_hbm, tbl_hbm, out_hbm,
             catv, idx, posv, gbufA, gbufB, outvA, outvB, tbl_sh,
             gsemA, gsemB, osemA, osemB, psem):
    wid = lax.axis_index("s") * NC + lax.axis_index("c")

    @pl.when(lax.axis_index("s") == 0)
    def _():
        pltpu.sync_copy(tbl_hbm, tbl_sh)

    plsc.subcore_barrier()

    def issue_gathers(ch, gbuf, gsem):
        base = ch * CHUNK
        for k in range(N_GATHER):
            pltpu.async_copy(
                tbl_sh.at[idx.at[pl.ds(k * N + base, CHUNK)]],
                gbuf.at[pl.ds(k * CHUNK, CHUNK)], gsem)

    def wait_gathers(gbuf, gsem):
        pltpu.make_async_copy(
            tbl_hbm.at[pl.ds(0, N_GATHER * CHUNK)], gbuf, gsem).wait()

    def sum_chunk(s, ch, gbuf, outv):
        pos_regs = [posv[pl.ds(i * 16, 16)] for i in range(D_MODEL // 16)]

        def tree(bs):
            while len(bs) > 1:
                bs = ([bs[i] + bs[i + 1] for i in range(0, len(bs) - 1, 2)]
                      + ([bs[-1]] if len(bs) % 2 else []))
            return bs[0]

        def do_tok(t, _):
            for wg in range(0, D_WORDS // 16, 2):
                bsA = [plsc.bitcast(gbuf[k * CHUNK + t, pl.ds(wg * 16, 16)],
                                    jnp.bfloat16) for k in range(N_GATHER)]
                bsB = [plsc.bitcast(gbuf[k * CHUNK + t, pl.ds(wg * 16 + 16, 16)],
                                    jnp.bfloat16) for k in range(N_GATHER)]
                for h, bs in ((wg, bsA), (wg + 1, bsB)):
                    xw = plsc.bitcast(tree(bs), jnp.int32)
                    lo = plsc.bitcast(xw << 16, jnp.float32)
                    hi = plsc.bitcast(xw & jnp.int32(-65536), jnp.float32)
                    outv[t, pl.ds(h * 32, 16)] = lo + pos_regs[2 * h]
                    outv[t, pl.ds(h * 32 + 16, 16)] = hi + pos_regs[2 * h + 1]
            return 0

        lax.fori_loop(0, CHUNK, do_tok, 0)

    def drain_out(outv, osem):
        pltpu.make_async_copy(outv, out_hbm.at[pl.ds(0, CHUNK)], osem).wait()

    def do_row(r, _):
        s = wid * ROWS_PER_W + r

        @pl.when(r > 0)
        def _():
            pltpu.make_async_copy(
                cat_hbm.at[pl.ds(0, CAT_L)], catv, psem).wait()

        for g in range(N // 16):
            sl = pl.ds(g * 16, 16)
            idx[sl] = catv[sl] * 7 + catv[pl.ds(N + g * 16, 16)]
            for j in range(N_ARGS):
                idx[pl.ds((1 + j) * N + g * 16, 16)] = (
                    catv[pl.ds((2 + j) * N + g * 16, 16)]
                    + (VC_ROWS + TBL_STRIDE * j + 1))
        for i in range(D_MODEL // 16):
            posv[pl.ds(i * 16, 16)] = plsc.bitcast(
                catv[pl.ds(13 * N + i * 16, 16)], jnp.float32)

        @pl.when(r < ROWS_PER_W - 1)
        def _():
            pltpu.async_copy(
                cat_hbm.at[pl.ds((s + 1) * CAT_L, CAT_L)], catv, psem)

        issue_gathers(0, gbufA, gsemA)

        def do_pair(i, _):
            issue_gathers(2 * i + 1, gbufB, gsemB)
            wait_gathers(gbufA, gsemA)

            @pl.when(i > 0)
            def _():
                drain_out(outvA, osemA)

            sum_chunk(s, 2 * i, gbufA, outvA)
            pltpu.async_copy(
                outvA, out_hbm.at[pl.ds(s * N + 2 * i * CHUNK, CHUNK)], osemA)

            @pl.when(i < (N // CHUNK) // 2 - 1)
            def _():
                issue_gathers(2 * i + 2, gbufA, gsemA)

            wait_gathers(gbufB, gsemB)

            @pl.when(i > 0)
            def _():
                drain_out(outvB, osemB)

            sum_chunk(s, 2 * i + 1, gbufB, outvB)
            pltpu.async_copy(
                outvB, out_hbm.at[pl.ds(s * N + (2 * i + 1) * CHUNK, CHUNK)],
                osemB)
            return 0

        lax.fori_loop(0, (N // CHUNK) // 2, do_pair, 0)
        drain_out(outvA, osemA)
        drain_out(outvB, osemB)
        return 0

    pltpu.sync_copy(
        cat_hbm.at[pl.ds(wid * ROWS_PER_W * CAT_L, CAT_L)], catv)
    lax.fori_loop(0, ROWS_PER_W, do_row, 0)


def _sc_bag(cat, tbl):
    mesh = plsc.VectorSubcoreMesh(core_axis_name="c", subcore_axis_name="s")
    return pl.kernel(
        _sc_body,
        out_type=jax.ShapeDtypeStruct((S * N, D_MODEL), jnp.float32),
        mesh=mesh,
        compiler_params=pltpu.CompilerParams(needs_layout_passes=False),
        scratch_types=[
            pltpu.VMEM((CAT_L,), jnp.int32),
            pltpu.VMEM((N_GATHER * N,), jnp.int32),
            pltpu.VMEM((D_MODEL,), jnp.float32),
            pltpu.VMEM((N_GATHER * CHUNK, D_WORDS), jnp.int32),
            pltpu.VMEM((N_GATHER * CHUNK, D_WORDS), jnp.int32),
            pltpu.VMEM((CHUNK, D_MODEL), jnp.float32),
            pltpu.VMEM((CHUNK, D_MODEL), jnp.float32),
            pltpu.VMEM_SHARED((TBL_ROWS, D_WORDS), jnp.int32),
            pltpu.SemaphoreType.DMA,
            pltpu.SemaphoreType.DMA,
            pltpu.SemaphoreType.DMA,
            pltpu.SemaphoreType.DMA,
            pltpu.SemaphoreType.DMA,
        ],
    )(cat, tbl)


def kernel(view, command, args, view_embed_w, command_embed_w, args_embed_w,
           args_mlp_w, args_mlp_b, mlp_w, mlp_b, pos_embed_w):
    tbl, pos = _build_tables(view_embed_w, command_embed_w, args_embed_w,
                             args_mlp_w, args_mlp_b, mlp_w, mlp_b, pos_embed_w)
    cat = jnp.concatenate(
        [view[:, None, :], command[:, None, :], args.transpose(0, 2, 1)],
        axis=1).reshape(S, 13 * N)
    cat = jnp.concatenate(
        [cat, lax.bitcast_convert_type(pos, jnp.int32)], axis=1).reshape(-1)
    tbl_i32 = lax.bitcast_convert_type(
        tbl.reshape(TBL_ROWS, D_WORDS, 2), jnp.int32)
    out = _sc_bag(cat, tbl_i32)
    return out.reshape(S, N, D_MODEL)


# speedup vs baseline: 16.8959x; 1.0060x over previous
import functools

import jax
import jax.numpy as jnp
from jax import lax
from jax.experimental import pallas as pl
from jax.experimental.pallas import tpu as pltpu
from jax.experimental.pallas import tpu_sc as plsc

D_MODEL = 256
N_ARGS = 11
S, N = 512, 512
VC_ROWS = 32
TBL_STRIDE = 264
TBL_ROWS = VC_ROWS + N_ARGS * TBL_STRIDE
NC, NS = 2, 16
NW = NC * NS
ROWS_PER_W = S // NW
CHUNK = 16
N_GATHER = 12
D_WORDS = D_MODEL // 2


def _build_tables_body(view_w_ref, cmd_w_ref, args_w_ref, args_mlp_w_ref,
                       args_mlp_b_ref, mlp_w_ref, mlp_b_ref, pos_w_ref,
                       tbl_ref, pos_ref):
    f32 = jnp.float32
    hi = jax.lax.Precision.HIGHEST
    mw_v = mlp_w_ref[0:4, :]
    mw_c = mlp_w_ref[4:12, :]
    mw_a = mlp_w_ref[12:140, :]
    cc = lax.broadcasted_iota(jnp.int32, (D_MODEL, D_MODEL), 1)
    src = ((cc // 32) * 32) + ((cc % 2) * 16) + ((cc % 32) // 2)
    perm = (lax.broadcasted_iota(jnp.int32, (D_MODEL, D_MODEL), 0)
            == src).astype(f32)
    vt = jnp.dot(view_w_ref[...], mw_v, precision=hi)
    ct = jnp.dot(cmd_w_ref[...], mw_c, precision=hi)
    r = lax.broadcasted_iota(jnp.int32, (VC_ROWS, 8), 0)
    ohv = (r // 7 == lax.broadcasted_iota(jnp.int32, (VC_ROWS, 8), 1)).astype(f32)
    ohc = (r % 7 == lax.broadcasted_iota(jnp.int32, (VC_ROWS, 8), 1)).astype(f32)
    vt8 = jnp.concatenate([vt, jnp.zeros((4, D_MODEL), f32)], axis=0)
    ct8 = jnp.concatenate([ct, jnp.zeros((1, D_MODEL), f32)], axis=0)
    vc = (jnp.dot(ohv, vt8, precision=hi)
          + jnp.dot(ohc, ct8, precision=hi))
    tbl_ref[0:VC_ROWS, :] = jnp.dot(vc, perm, precision=hi).astype(jnp.bfloat16)
    for j in range(N_ARGS):
        m_j = jnp.dot(args_mlp_w_ref[64 * j:64 * (j + 1), :], mw_a, precision=hi)
        at_j = jnp.dot(args_w_ref[...], m_j, precision=hi)
        base = VC_ROWS + j * TBL_STRIDE
        tbl_ref[base:base + 257, :] = jnp.dot(at_j, perm, precision=hi).astype(jnp.bfloat16)
        tbl_ref[base + 257:base + TBL_STRIDE, :] = jnp.zeros(
            (TBL_STRIDE - 257, D_MODEL), jnp.bfloat16)
    bias = mlp_b_ref[...] + jnp.dot(args_mlp_b_ref[...], mw_a, precision=hi)
    pos_ref[...] = pos_w_ref[0:S, :] + bias


def _build_tables(view_w, cmd_w, args_w, args_mlp_w, args_mlp_b, mlp_w, mlp_b, pos_w):
    return pl.pallas_call(
        _build_tables_body,
        out_shape=(
            jax.ShapeDtypeStruct((TBL_ROWS, D_MODEL), jnp.bfloat16),
            jax.ShapeDtypeStruct((S, D_MODEL), jnp.float32),
        ),
    )(view_w, cmd_w, args_w, args_mlp_w,
      args_mlp_b.reshape(1, -1), mlp_w, mlp_b.reshape(1, -1), pos_w)


CAT_L = 13 * N + D_MODEL


def _sc_body(cat_hbm, tbl_hbm, out_hbm,
             catv, idx, posv, gbufA, gbufB, outvA, outvB, tbl_sh,
             gsemA, gsemB, osemA, osemB, psem):
    wid = lax.axis_index("s") * NC + lax.axis_index("c")

    @pl.when(lax.axis_index("s") == 0)
    def _():
        pltpu.sync_copy(tbl_hbm, tbl_sh)

    plsc.subcore_barrier()

    def issue_gathers(ch, gbuf, gsem):
        st = ch * N_GATHER * CHUNK
        half = N_GATHER * CHUNK // 2
        pltpu.async_copy(tbl_sh.at[idx.at[pl.ds(st, half)]],
                         gbuf.at[pl.ds(0, half)], gsem)
        pltpu.async_copy(tbl_sh.at[idx.at[pl.ds(st + half, half)]],
                         gbuf.at[pl.ds(half, half)], gsem)

    def wait_gathers(gbuf, gsem):
        pltpu.make_async_copy(
            tbl_hbm.at[pl.ds(0, N_GATHER * CHUNK)], gbuf, gsem).wait()

    def sum_chunk(s, ch, gbuf, outv):
        pos_regs = [posv[pl.ds(i * 16, 16)] for i in range(D_MODEL // 16)]

        def tree(bs):
            while len(bs) > 1:
                bs = ([bs[i] + bs[i + 1] for i in range(0, len(bs) - 1, 2)]
                      + ([bs[-1]] if len(bs) % 2 else []))
            return bs[0]

        def do_tok(t, _):
            for wg in range(0, D_WORDS // 16, 2):
                bsA = [plsc.bitcast(gbuf[k * CHUNK + t, pl.ds(wg * 16, 16)],
                                    jnp.bfloat16) for k in range(N_GATHER)]
                bsB = [plsc.bitcast(gbuf[k * CHUNK + t, pl.ds(wg * 16 + 16, 16)],
                                    jnp.bfloat16) for k in range(N_GATHER)]
                for h, bs in ((wg, bsA), (wg + 1, bsB)):
                    xw = plsc.bitcast(tree(bs), jnp.int32)
                    lo = plsc.bitcast(xw << 16, jnp.float32)
                    hi = plsc.bitcast(xw & jnp.int32(-65536), jnp.float32)
                    outv[t, pl.ds(h * 32, 16)] = lo + pos_regs[2 * h]
                    outv[t, pl.ds(h * 32 + 16, 16)] = hi + pos_regs[2 * h + 1]
            return 0

        lax.fori_loop(0, CHUNK, do_tok, 0)

    def drain_out(outv, osem):
        pltpu.make_async_copy(outv, out_hbm.at[pl.ds(0, CHUNK)], osem).wait()

    def do_row(r, _):
        s = wid * ROWS_PER_W + r

        @pl.when(r > 0)
        def _():
            pltpu.make_async_copy(
                cat_hbm.at[pl.ds(0, CAT_L)], catv, psem).wait()

        for g in range(N // 16):
            base = g * N_GATHER * 16
            sl = pl.ds(g * 16, 16)
            idx[pl.ds(base, 16)] = catv[sl] * 7 + catv[pl.ds(N + g * 16, 16)]
            for j in range(N_ARGS):
                idx[pl.ds(base + (1 + j) * 16, 16)] = (
                    catv[pl.ds((2 + j) * N + g * 16, 16)]
                    + (VC_ROWS + TBL_STRIDE * j + 1))
        for i in range(D_MODEL // 16):
            posv[pl.ds(i * 16, 16)] = plsc.bitcast(
                catv[pl.ds(13 * N + i * 16, 16)], jnp.float32)

        @pl.when(r < ROWS_PER_W - 1)
        def _():
            pltpu.async_copy(
                cat_hbm.at[pl.ds((s + 1) * CAT_L, CAT_L)], catv, psem)

        issue_gathers(0, gbufA, gsemA)

        def do_pair(i, _):
            issue_gathers(2 * i + 1, gbufB, gsemB)
            wait_gathers(gbufA, gsemA)

            @pl.when(i > 0)
            def _():
                drain_out(outvA, osemA)

            sum_chunk(s, 2 * i, gbufA, outvA)
            pltpu.async_copy(
                outvA, out_hbm.at[pl.ds(s * N + 2 * i * CHUNK, CHUNK)], osemA)

            @pl.when(i < (N // CHUNK) // 2 - 1)
            def _():
                issue_gathers(2 * i + 2, gbufA, gsemA)

            wait_gathers(gbufB, gsemB)

            @pl.when(i > 0)
            def _():
                drain_out(outvB, osemB)

            sum_chunk(s, 2 * i + 1, gbufB, outvB)
            pltpu.async_copy(
                outvB, out_hbm.at[pl.ds(s * N + (2 * i + 1) * CHUNK, CHUNK)],
                osemB)
            return 0

        lax.fori_loop(0, (N // CHUNK) // 2, do_pair, 0)
        drain_out(outvA, osemA)
        drain_out(outvB, osemB)
        return 0

    pltpu.sync_copy(
        cat_hbm.at[pl.ds(wid * ROWS_PER_W * CAT_L, CAT_L)], catv)
    lax.fori_loop(0, ROWS_PER_W, do_row, 0)


def _sc_bag(cat, tbl):
    mesh = plsc.VectorSubcoreMesh(core_axis_name="c", subcore_axis_name="s")
    return pl.kernel(
        _sc_body,
        out_type=jax.ShapeDtypeStruct((S * N, D_MODEL), jnp.float32),
        mesh=mesh,
        compiler_params=pltpu.CompilerParams(needs_layout_passes=False),
        scratch_types=[
            pltpu.VMEM((CAT_L,), jnp.int32),
            pltpu.VMEM((N_GATHER * N,), jnp.int32),
            pltpu.VMEM((D_MODEL,), jnp.float32),
            pltpu.VMEM((N_GATHER * CHUNK, D_WORDS), jnp.int32),
            pltpu.VMEM((N_GATHER * CHUNK, D_WORDS), jnp.int32),
            pltpu.VMEM((CHUNK, D_MODEL), jnp.float32),
            pltpu.VMEM((CHUNK, D_MODEL), jnp.float32),
            pltpu.VMEM_SHARED((TBL_ROWS, D_WORDS), jnp.int32),
            pltpu.SemaphoreType.DMA,
            pltpu.SemaphoreType.DMA,
            pltpu.SemaphoreType.DMA,
            pltpu.SemaphoreType.DMA,
            pltpu.SemaphoreType.DMA,
        ],
    )(cat, tbl)


def kernel(view, command, args, view_embed_w, command_embed_w, args_embed_w,
           args_mlp_w, args_mlp_b, mlp_w, mlp_b, pos_embed_w):
    tbl, pos = _build_tables(view_embed_w, command_embed_w, args_embed_w,
                             args_mlp_w, args_mlp_b, mlp_w, mlp_b, pos_embed_w)
    cat = jnp.concatenate(
        [view[:, None, :], command[:, None, :], args.transpose(0, 2, 1)],
        axis=1).reshape(S, 13 * N)
    cat = jnp.concatenate(
        [cat, lax.bitcast_convert_type(pos, jnp.int32)], axis=1).reshape(-1)
    tbl_i32 = lax.bitcast_convert_type(
        tbl.reshape(TBL_ROWS, D_WORDS, 2), jnp.int32)
    out = _sc_bag(cat, tbl_i32)
    return out.reshape(S, N, D_MODEL)


# speedup vs baseline: 19.4040x; 1.1484x over previous
import functools

import jax
import jax.numpy as jnp
from jax import lax
from jax.experimental import pallas as pl
from jax.experimental.pallas import tpu as pltpu
from jax.experimental.pallas import tpu_sc as plsc

D_MODEL = 256
N_ARGS = 11
S, N = 512, 512
VC_ROWS = 32
TBL_STRIDE = 264
TBL_ROWS = VC_ROWS + N_ARGS * TBL_STRIDE
NC, NS = 2, 16
NW = NC * NS
ROWS_PER_W = S // NW
CHUNK = 16
N_GATHER = 12
D_WORDS = D_MODEL // 2


def _build_tables_body(view_w_ref, cmd_w_ref, args_w_ref, args_mlp_w_ref,
                       args_mlp_b_ref, mlp_w_ref, mlp_b_ref, pos_w_ref,
                       tbl_ref, pos_ref):
    f32 = jnp.float32
    hi = jax.lax.Precision.HIGHEST
    mw_v = mlp_w_ref[0:4, :]
    mw_c = mlp_w_ref[4:12, :]
    mw_a = mlp_w_ref[12:140, :]
    cc = lax.broadcasted_iota(jnp.int32, (D_MODEL, D_MODEL), 1)
    src = ((cc // 32) * 32) + ((cc % 2) * 16) + ((cc % 32) // 2)
    perm = (lax.broadcasted_iota(jnp.int32, (D_MODEL, D_MODEL), 0)
            == src).astype(f32)
    vt = jnp.dot(view_w_ref[...], mw_v, precision=hi)
    ct = jnp.dot(cmd_w_ref[...], mw_c, precision=hi)
    r = lax.broadcasted_iota(jnp.int32, (VC_ROWS, 8), 0)
    ohv = (r // 7 == lax.broadcasted_iota(jnp.int32, (VC_ROWS, 8), 1)).astype(f32)
    ohc = (r % 7 == lax.broadcasted_iota(jnp.int32, (VC_ROWS, 8), 1)).astype(f32)
    vt8 = jnp.concatenate([vt, jnp.zeros((4, D_MODEL), f32)], axis=0)
    ct8 = jnp.concatenate([ct, jnp.zeros((1, D_MODEL), f32)], axis=0)
    vc = (jnp.dot(ohv, vt8, precision=hi)
          + jnp.dot(ohc, ct8, precision=hi))
    tbl_ref[0:VC_ROWS, :] = jnp.dot(vc, perm, precision=hi).astype(jnp.bfloat16)
    for j in range(N_ARGS):
        m_j = jnp.dot(args_mlp_w_ref[64 * j:64 * (j + 1), :], mw_a, precision=hi)
        at_j = jnp.dot(args_w_ref[...], m_j, precision=hi)
        base = VC_ROWS + j * TBL_STRIDE
        tbl_ref[base:base + 257, :] = jnp.dot(at_j, perm, precision=hi).astype(jnp.bfloat16)
        tbl_ref[base + 257:base + TBL_STRIDE, :] = jnp.zeros(
            (TBL_STRIDE - 257, D_MODEL), jnp.bfloat16)
    bias = mlp_b_ref[...] + jnp.dot(args_mlp_b_ref[...], mw_a, precision=hi)
    pos_ref[...] = pos_w_ref[0:S, :] + bias


def _build_tables(view_w, cmd_w, args_w, args_mlp_w, args_mlp_b, mlp_w, mlp_b, pos_w):
    return pl.pallas_call(
        _build_tables_body,
        out_shape=(
            jax.ShapeDtypeStruct((TBL_ROWS, D_MODEL), jnp.bfloat16),
            jax.ShapeDtypeStruct((S, D_MODEL), jnp.float32),
        ),
    )(view_w, cmd_w, args_w, args_mlp_w,
      args_mlp_b.reshape(1, -1), mlp_w, mlp_b.reshape(1, -1), pos_w)


CAT_L = 13 * N + D_MODEL


def _sc_body(cat_hbm, tbl_hbm, out_hbm,
             catv, idx, posv, gbufA, gbufB, outvA, outvB, tbl_sh,
             gsemA, gsemB, osemA, osemB, psem):
    wid = lax.axis_index("s") * NC + lax.axis_index("c")

    @pl.when(lax.axis_index("s") == 0)
    def _():
        pltpu.sync_copy(tbl_hbm, tbl_sh)

    plsc.subcore_barrier()

    def issue_gathers(ch, gbuf, gsem):
        st = ch * N_GATHER * CHUNK
        half = N_GATHER * CHUNK // 2
        pltpu.async_copy(tbl_sh.at[idx.at[pl.ds(st, half)]],
                         gbuf.at[pl.ds(0, half)], gsem)
        pltpu.async_copy(tbl_sh.at[idx.at[pl.ds(st + half, half)]],
                         gbuf.at[pl.ds(half, half)], gsem)

    def wait_gathers(gbuf, gsem):
        pltpu.make_async_copy(
            tbl_hbm.at[pl.ds(0, N_GATHER * CHUNK)], gbuf, gsem).wait()

    def sum_chunk(s, ch, gbuf, outv):
        pos_regs = [posv[pl.ds(i * 16, 16)] for i in range(D_MODEL // 16)]

        def tree(bs):
            while len(bs) > 1:
                bs = ([bs[i] + bs[i + 1] for i in range(0, len(bs) - 1, 2)]
                      + ([bs[-1]] if len(bs) % 2 else []))
            return bs[0]

        @plsc.parallel_loop(0, CHUNK, 1)
        def do_tok(t):
            for wg in range(0, D_WORDS // 16, 2):
                bsA = [plsc.bitcast(gbuf[k * CHUNK + t, pl.ds(wg * 16, 16)],
                                    jnp.bfloat16) for k in range(N_GATHER)]
                bsB = [plsc.bitcast(gbuf[k * CHUNK + t, pl.ds(wg * 16 + 16, 16)],
                                    jnp.bfloat16) for k in range(N_GATHER)]
                for h, bs in ((wg, bsA), (wg + 1, bsB)):
                    xw = plsc.bitcast(tree(bs), jnp.int32)
                    lo = plsc.bitcast(xw << 16, jnp.float32)
                    hi = plsc.bitcast(xw & jnp.int32(-65536), jnp.float32)
                    outv[t, pl.ds(h * 32, 16)] = lo + pos_regs[2 * h]
                    outv[t, pl.ds(h * 32 + 16, 16)] = hi + pos_regs[2 * h + 1]

    def drain_out(outv, osem):
        pltpu.make_async_copy(outv, out_hbm.at[pl.ds(0, CHUNK)], osem).wait()

    def do_row(r, _):
        s = wid * ROWS_PER_W + r

        @pl.when(r > 0)
        def _():
            pltpu.make_async_copy(
                cat_hbm.at[pl.ds(0, CAT_L)], catv, psem).wait()

        for g in range(N // 16):
            base = g * N_GATHER * 16
            sl = pl.ds(g * 16, 16)
            idx[pl.ds(base, 16)] = catv[sl] * 7 + catv[pl.ds(N + g * 16, 16)]
            for j in range(N_ARGS):
                idx[pl.ds(base + (1 + j) * 16, 16)] = (
                    catv[pl.ds((2 + j) * N + g * 16, 16)]
                    + (VC_ROWS + TBL_STRIDE * j + 1))
        for i in range(D_MODEL // 16):
            posv[pl.ds(i * 16, 16)] = plsc.bitcast(
                catv[pl.ds(13 * N + i * 16, 16)], jnp.float32)

        @pl.when(r < ROWS_PER_W - 1)
        def _():
            pltpu.async_copy(
                cat_hbm.at[pl.ds((s + 1) * CAT_L, CAT_L)], catv, psem)

        issue_gathers(0, gbufA, gsemA)

        def do_pair(i, _):
            issue_gathers(2 * i + 1, gbufB, gsemB)
            wait_gathers(gbufA, gsemA)

            @pl.when(i > 0)
            def _():
                drain_out(outvA, osemA)

            sum_chunk(s, 2 * i, gbufA, outvA)
            pltpu.async_copy(
                outvA, out_hbm.at[pl.ds(s * N + 2 * i * CHUNK, CHUNK)], osemA)

            @pl.when(i < (N // CHUNK) // 2 - 1)
            def _():
                issue_gathers(2 * i + 2, gbufA, gsemA)

            wait_gathers(gbufB, gsemB)

            @pl.when(i > 0)
            def _():
                drain_out(outvB, osemB)

            sum_chunk(s, 2 * i + 1, gbufB, outvB)
            pltpu.async_copy(
                outvB, out_hbm.at[pl.ds(s * N + (2 * i + 1) * CHUNK, CHUNK)],
                osemB)
            return 0

        lax.fori_loop(0, (N // CHUNK) // 2, do_pair, 0)
        drain_out(outvA, osemA)
        drain_out(outvB, osemB)
        return 0

    pltpu.sync_copy(
        cat_hbm.at[pl.ds(wid * ROWS_PER_W * CAT_L, CAT_L)], catv)
    lax.fori_loop(0, ROWS_PER_W, do_row, 0)


def _sc_bag(cat, tbl):
    mesh = plsc.VectorSubcoreMesh(core_axis_name="c", subcore_axis_name="s")
    return pl.kernel(
        _sc_body,
        out_type=jax.ShapeDtypeStruct((S * N, D_MODEL), jnp.float32),
        mesh=mesh,
        compiler_params=pltpu.CompilerParams(needs_layout_passes=False),
        scratch_types=[
            pltpu.VMEM((CAT_L,), jnp.int32),
            pltpu.VMEM((N_GATHER * N,), jnp.int32),
            pltpu.VMEM((D_MODEL,), jnp.float32),
            pltpu.VMEM((N_GATHER * CHUNK, D_WORDS), jnp.int32),
            pltpu.VMEM((N_GATHER * CHUNK, D_WORDS), jnp.int32),
            pltpu.VMEM((CHUNK, D_MODEL), jnp.float32),
            pltpu.VMEM((CHUNK, D_MODEL), jnp.float32),
            pltpu.VMEM_SHARED((TBL_ROWS, D_WORDS), jnp.int32),
            pltpu.SemaphoreType.DMA,
            pltpu.SemaphoreType.DMA,
            pltpu.SemaphoreType.DMA,
            pltpu.SemaphoreType.DMA,
            pltpu.SemaphoreType.DMA,
        ],
    )(cat, tbl)


def kernel(view, command, args, view_embed_w, command_embed_w, args_embed_w,
           args_mlp_w, args_mlp_b, mlp_w, mlp_b, pos_embed_w):
    tbl, pos = _build_tables(view_embed_w, command_embed_w, args_embed_w,
                             args_mlp_w, args_mlp_b, mlp_w, mlp_b, pos_embed_w)
    cat = jnp.concatenate(
        [view[:, None, :], command[:, None, :], args.transpose(0, 2, 1)],
        axis=1).reshape(S, 13 * N)
    cat = jnp.concatenate(
        [cat, lax.bitcast_convert_type(pos, jnp.int32)], axis=1).reshape(-1)
    tbl_i32 = lax.bitcast_convert_type(
        tbl.reshape(TBL_ROWS, D_WORDS, 2), jnp.int32)
    out = _sc_bag(cat, tbl_i32)
    return out.reshape(S, N, D_MODEL)
